# baseline (speedup 1.0000x reference)
import jax
import jax.numpy as jnp
from jax import lax
from jax.experimental import pallas as pl
from jax.experimental.pallas import tpu as pltpu

_H = 32
_W = 32
_F = 128
_HW = _H * _W


def _pos_body(colT_ref, rowT_ref, out_ref, pos_ref):
    @pl.when(pl.program_id(0) == 0)
    def _init():
        hw = lax.broadcasted_iota(jnp.int32, (_W, _HW), 1)
        sel_w = lax.broadcasted_iota(jnp.int32, (_W, _HW), 0)
        S = (lax.rem(hw, _W) == sel_w).astype(jnp.float32)
        R = (lax.div(hw, _W) == sel_w).astype(jnp.float32)
        pos_ref[:_F, :] = jnp.dot(
            colT_ref[...], S, preferred_element_type=jnp.float32,
            precision=lax.Precision.HIGHEST,
        )
        pos_ref[_F:, :] = jnp.dot(
            rowT_ref[...], R, preferred_element_type=jnp.float32,
            precision=lax.Precision.HIGHEST,
        )

    out_ref[0] = pos_ref[...]


def kernel(mask, row_weight, col_weight):
    bs, h, w = mask.shape
    colT = col_weight[:w].T
    rowT = row_weight[:h].T

    out = pl.pallas_call(
        _pos_body,
        grid=(bs,),
        in_specs=[
            pl.BlockSpec((_F, _W), lambda b: (0, 0)),
            pl.BlockSpec((_F, _H), lambda b: (0, 0)),
        ],
        out_specs=pl.BlockSpec((1, 2 * _F, _HW), lambda b: (b, 0, 0)),
        out_shape=jax.ShapeDtypeStruct((bs, 2 * _F, _HW), jnp.float32),
        scratch_shapes=[pltpu.VMEM((2 * _F, _HW), jnp.float32)],
    )(colT, rowT)
    return out.reshape(bs, 2 * _F, h, w)


# speedup vs baseline: 1.0657x; 1.0657x over previous
import jax
import jax.numpy as jnp
from jax import lax
from jax.experimental import pallas as pl
from jax.experimental.pallas import tpu as pltpu

_H = 32
_W = 32
_F = 128
_HW = _H * _W
_BS = 16


def _pos_body(colT_ref, rowT_ref, out_hbm, pos_ref, sem):
    hw = lax.broadcasted_iota(jnp.int32, (_W, _HW), 1)
    sel = lax.broadcasted_iota(jnp.int32, (_W, _HW), 0)
    S = (lax.rem(hw, _W) == sel).astype(jnp.float32)
    R = (lax.div(hw, _W) == sel).astype(jnp.float32)
    pos_ref[:_F, :] = jnp.dot(
        colT_ref[...], S, preferred_element_type=jnp.float32,
        precision=lax.Precision.HIGHEST,
    )
    pos_ref[_F:, :] = jnp.dot(
        rowT_ref[...], R, preferred_element_type=jnp.float32,
        precision=lax.Precision.HIGHEST,
    )
    copies = [
        pltpu.make_async_copy(pos_ref, out_hbm.at[b], sem) for b in range(_BS)
    ]
    for c in copies:
        c.start()
    for c in copies:
        c.wait()


def kernel(mask, row_weight, col_weight):
    bs, h, w = mask.shape
    colT = col_weight[:w].T
    rowT = row_weight[:h].T

    out = pl.pallas_call(
        _pos_body,
        in_specs=[
            pl.BlockSpec((_F, _W), lambda: (0, 0)),
            pl.BlockSpec((_F, _H), lambda: (0, 0)),
        ],
        out_specs=pl.BlockSpec(memory_space=pl.ANY),
        out_shape=jax.ShapeDtypeStruct((bs, 2 * _F, _HW), jnp.float32),
        scratch_shapes=[
            pltpu.VMEM((2 * _F, _HW), jnp.float32),
            pltpu.SemaphoreType.DMA,
        ],
    )(colT, rowT)
    return out.reshape(bs, 2 * _F, h, w)


# speedup vs baseline: 1.0667x; 1.0010x over previous
import jax
import jax.numpy as jnp
from jax import lax
from jax.experimental import pallas as pl
from jax.experimental.pallas import tpu as pltpu

_H = 32
_W = 32
_F = 128
_HW = _H * _W
_BS = 16


def _pos_body(colT_ref, rowT_ref, out_hbm, pos_ref, sem):
    hw = lax.broadcasted_iota(jnp.int32, (_W, _HW), 1)
    sel = lax.broadcasted_iota(jnp.int32, (_W, _HW), 0)
    S = (lax.rem(hw, _W) == sel).astype(jnp.float32)
    R = (lax.div(hw, _W) == sel).astype(jnp.float32)
    pos_ref[:_F, :] = jnp.dot(
        colT_ref[...], S, preferred_element_type=jnp.float32,
        precision=lax.Precision.HIGHEST,
    )
    pos_ref[_F:, :] = jnp.dot(
        rowT_ref[...], R, preferred_element_type=jnp.float32,
        precision=lax.Precision.HIGHEST,
    )
    copies = [
        pltpu.make_async_copy(pos_ref, out_hbm.at[b], sem.at[b])
        for b in range(_BS)
    ]
    for c in copies:
        c.start()
    for c in copies:
        c.wait()


def kernel(mask, row_weight, col_weight):
    bs, h, w = mask.shape
    colT = col_weight[:w].T
    rowT = row_weight[:h].T

    out = pl.pallas_call(
        _pos_body,
        in_specs=[
            pl.BlockSpec((_F, _W), lambda: (0, 0)),
            pl.BlockSpec((_F, _H), lambda: (0, 0)),
        ],
        out_specs=pl.BlockSpec(memory_space=pl.ANY),
        out_shape=jax.ShapeDtypeStruct((bs, 2 * _F, _HW), jnp.float32),
        scratch_shapes=[
            pltpu.VMEM((2 * _F, _HW), jnp.float32),
            pltpu.SemaphoreType.DMA((_BS,)),
        ],
    )(colT, rowT)
    return out.reshape(bs, 2 * _F, h, w)
